# Initial kernel scaffold; baseline (speedup 1.0000x reference)
import jax
import jax.numpy as jnp
from jax.experimental import pallas as pl


def kernel(indices, table):
    raise NotImplementedError("write your pallas kernel here")


# speedup vs baseline: 4.5914x; 4.5914x over previous
import functools

import jax
import jax.numpy as jnp
from jax import lax
from jax.experimental import pallas as pl
from jax.experimental.pallas import tpu as pltpu
from jax.experimental.pallas import tpu_sc as plsc

NUM_ROWS = 4096 * 50
DIM = 64
NUM_WORKERS = 32
ROWS_PER_WORKER = NUM_ROWS // NUM_WORKERS
CHUNK = 800
NUM_CHUNKS = ROWS_PER_WORKER // CHUNK


def _gather_kernel(idx_hbm, table_hbm, out_hbm, idx_v, rows_v, sem):
    wid = lax.axis_index("s") * 2 + lax.axis_index("c")
    base = wid * ROWS_PER_WORKER
    pltpu.sync_copy(idx_hbm.at[pl.ds(base, ROWS_PER_WORKER)], idx_v)
    for j in range(NUM_CHUNKS):
        pltpu.async_copy(
            table_hbm.at[idx_v.at[pl.ds(j * CHUNK, CHUNK)]], rows_v, sem
        ).wait()
        pltpu.sync_copy(rows_v, out_hbm.at[pl.ds(base + j * CHUNK, CHUNK)])


@jax.jit
def _lookup(idx_flat, table):
    mesh = plsc.VectorSubcoreMesh(core_axis_name="c", subcore_axis_name="s")
    run = functools.partial(
        pl.kernel,
        out_type=jax.ShapeDtypeStruct((NUM_ROWS, DIM), jnp.float32),
        mesh=mesh,
        scratch_types=[
            pltpu.VMEM((ROWS_PER_WORKER,), jnp.int32),
            pltpu.VMEM((CHUNK, DIM), jnp.float32),
            pltpu.SemaphoreType.DMA,
        ],
        compiler_params=pltpu.CompilerParams(use_tc_tiling_on_sc=False),
    )(_gather_kernel)
    return run(idx_flat, table)


def kernel(indices, table):
    b, l = indices.shape
    idx_flat = indices.reshape(-1).astype(jnp.int32)
    out = _lookup(idx_flat, table)
    return out.reshape(b, l, DIM)


# speedup vs baseline: 4.6629x; 1.0156x over previous
import functools

import jax
import jax.numpy as jnp
from jax import lax
from jax.experimental import pallas as pl
from jax.experimental.pallas import tpu as pltpu
from jax.experimental.pallas import tpu_sc as plsc

NUM_ROWS = 4096 * 50
DIM = 64
NUM_WORKERS = 32
ROWS_PER_WORKER = NUM_ROWS // NUM_WORKERS
CHUNK = 800
NUM_CHUNKS = ROWS_PER_WORKER // CHUNK


def _gather_kernel(idx_hbm, table_hbm, out_hbm, idx_v, rows0, rows1,
                   gsem0, gsem1, wsem0, wsem1):
    wid = lax.axis_index("s") * 2 + lax.axis_index("c")
    base = wid * ROWS_PER_WORKER
    rows = (rows0, rows1)
    gsem = (gsem0, gsem1)
    wsem = (wsem0, wsem1)
    pltpu.sync_copy(idx_hbm.at[pl.ds(base, ROWS_PER_WORKER)], idx_v)

    def start_gather(j):
        return pltpu.async_copy(
            table_hbm.at[idx_v.at[pl.ds(j * CHUNK, CHUNK)]],
            rows[j % 2], gsem[j % 2])

    writes = [None, None]
    gathers = [None, None]
    gathers[0] = start_gather(0)
    for j in range(NUM_CHUNKS):
        b = j % 2
        if j + 1 < NUM_CHUNKS:
            nb = (j + 1) % 2
            if writes[nb] is not None:
                writes[nb].wait()
            gathers[nb] = start_gather(j + 1)
        gathers[b].wait()
        writes[b] = pltpu.async_copy(
            rows[b], out_hbm.at[pl.ds(base + j * CHUNK, CHUNK)], wsem[b])
    writes[0].wait()
    writes[1].wait()


@jax.jit
def _lookup(idx_flat, table):
    mesh = plsc.VectorSubcoreMesh(core_axis_name="c", subcore_axis_name="s")
    run = functools.partial(
        pl.kernel,
        out_type=jax.ShapeDtypeStruct((NUM_ROWS, DIM), jnp.float32),
        mesh=mesh,
        scratch_types=[
            pltpu.VMEM((ROWS_PER_WORKER,), jnp.int32),
            pltpu.VMEM((CHUNK, DIM), jnp.float32),
            pltpu.VMEM((CHUNK, DIM), jnp.float32),
            pltpu.SemaphoreType.DMA,
            pltpu.SemaphoreType.DMA,
            pltpu.SemaphoreType.DMA,
            pltpu.SemaphoreType.DMA,
        ],
        compiler_params=pltpu.CompilerParams(use_tc_tiling_on_sc=False),
    )(_gather_kernel)
    return run(idx_flat, table)


def kernel(indices, table):
    b, l = indices.shape
    idx_flat = indices.reshape(-1).astype(jnp.int32)
    out = _lookup(idx_flat, table)
    return out.reshape(b, l, DIM)
